# baseline (speedup 1.0000x reference)
import functools

import jax
import jax.numpy as jnp
from jax import lax
from jax.experimental import pallas as pl
from jax.experimental.pallas import tpu as pltpu
from jax.experimental.pallas import tpu_sc as plsc

N_NODES = 10000
N_EDGES = 320000
DIM = 128

NC = 2
NS = 16
NW = NC * NS
EPT = N_EDGES // NW
C = 80
NCHUNK = EPT // C
RPT = N_NODES // NS
ZB = 125

MM_BLK = 1000


def _tc_mm_body(x_ref, w_ref, h_ref):
    h_ref[...] = jnp.dot(x_ref[...], w_ref[...],
                         preferred_element_type=jnp.float32)


def _tc_score_body(h_ref, a2_ref, s_ref):
    s_ref[...] = lax.dot_general(a2_ref[...], h_ref[...],
                                 (((0,), (1,)), ((), ())),
                                 preferred_element_type=jnp.float32)


def _tc_fin_body(agg_ref, den_ref, o_ref):
    a = agg_ref[0] + agg_ref[1]
    dsum = jnp.sum(den_ref[...], axis=0)
    o_ref[...] = jnp.maximum(a / (dsum[:, None] + 1e-16), 0.0)


ASC = 5
BPS = NCHUNK // ASC


def _sc_a_body(s2t_hbm, eia_hbm, w_hbm, den_hbm,
               ssrc_v, sdst_v, den_v, eidx_v, wv_v):
    c = lax.axis_index("c")
    s = lax.axis_index("s")
    wid = s * NC + c

    pltpu.sync_copy(s2t_hbm.at[0], ssrc_v)
    pltpu.sync_copy(s2t_hbm.at[1], sdst_v)

    def _zden(i, _):
        den_v[pl.ds(i * 16, 16)] = jnp.zeros((16,), jnp.float32)
        return _
    lax.fori_loop(0, N_NODES // 16, _zden, 0)

    def _super(t, _):
        pltpu.sync_copy(eia_hbm.at[wid, pl.ds(t * BPS, BPS)], eidx_v)

        def _cc(cc, _c):
            def _grp(g, _g):
                base = g * 16
                s16 = eidx_v[cc, 0, pl.ds(base, 16)]
                d16 = eidx_v[cc, 1, pl.ds(base, 16)]
                x = (plsc.load_gather(ssrc_v, [s16])
                     + plsc.load_gather(sdst_v, [d16]))
                e = jnp.where(x >= 0, x, 0.2 * x)
                w16 = jnp.exp(e)
                wv_v[cc, pl.ds(base, 16)] = w16
                plsc.addupdate_scatter(den_v, [d16], w16)
                return _g
            lax.fori_loop(0, C // 16, _grp, 0)
            return _c
        lax.fori_loop(0, BPS, _cc, 0)
        pltpu.sync_copy(wv_v, w_hbm.at[wid, pl.ds(t * BPS, BPS)])
        return _
    lax.fori_loop(0, ASC, _super, 0)
    pltpu.sync_copy(den_v, den_hbm.at[wid])


def _sc_b_body(h_hbm, eib_hbm, wb_hbm, agg_hbm,
               rows0, rows1, rows2, rows3, idx0, idx1, idx2, idx3,
               wv0, wv1, wv2, wv3, agg_sh,
               g0, g1, g2, g3, i0, i1, i2, i3, s0, s1, s2, s3):
    c = lax.axis_index("c")
    s = lax.axis_index("s")
    wid = s * NC + c
    row0 = s * RPT

    rows = (rows0, rows1, rows2, rows3)
    idx = (idx0, idx1, idx2, idx3)
    wv = (wv0, wv1, wv2, wv3)
    gsem = (g0, g1, g2, g3)
    isem = (i0, i1, i2, i3)
    ssem = (s0, s1, s2, s3)

    def issue_idx(b, ch):
        pltpu.async_copy(eib_hbm.at[wid, ch], idx[b], isem[b])
        pltpu.async_copy(wb_hbm.at[wid, ch], wv[b], isem[b])

    def wait_idx(b, ch):
        pltpu.make_async_copy(eib_hbm.at[wid, ch], idx[b], isem[b]).wait()
        pltpu.make_async_copy(wb_hbm.at[wid, ch], wv[b], isem[b]).wait()

    def issue_gather(b):
        pltpu.async_copy(h_hbm.at[idx[b].at[0]], rows[b], gsem[b])

    def wait_gather(b):
        pltpu.make_async_copy(h_hbm.at[idx[b].at[0]], rows[b],
                              gsem[b]).wait()

    def issue_scatter(b):
        pltpu.async_copy(rows[b], agg_sh.at[idx[b].at[1]], ssem[b],
                         add=True)

    def drain_scatter(b):
        pltpu.make_async_copy(rows[b], agg_sh.at[idx[b].at[1]],
                              ssem[b]).wait()

    def compute(b):
        @plsc.parallel_loop(0, C // 16, 1, unroll=2)
        def _grp(g):
            base = g * 16
            w16 = wv[b][pl.ds(base, 16)]
            for l in range(16):
                wl = w16[l]
                for j in range(DIM // 16):
                    sl = pl.ds(j * 16, 16)
                    rows[b][base + l, sl] = rows[b][base + l, sl] * wl

    def _zrows(i, _):
        for j in range(DIM // 16):
            rows0[i, pl.ds(j * 16, 16)] = jnp.zeros((16,), jnp.float32)
        return _
    lax.fori_loop(0, C, _zrows, 0)

    def _zagg(k, _):
        pltpu.sync_copy(rows0, agg_sh.at[pl.ds(row0 + k * C, C)])
        return _
    lax.fori_loop(0, RPT // C, _zagg, 0)
    pltpu.sync_copy(rows0.at[pl.ds(0, RPT % C)],
                    agg_sh.at[pl.ds(row0 + (RPT // C) * C, RPT % C)])
    plsc.subcore_barrier()

    pltpu.sync_copy(eib_hbm.at[wid, 0], idx0)
    pltpu.sync_copy(wb_hbm.at[wid, 0], wv0)
    issue_gather(0)
    issue_idx(1, 1)
    issue_idx(2, 2)
    wait_idx(1, 1)
    issue_gather(1)

    def _quad(t, _):
        for k in range(4):
            ch = t * 4 + k
            wait_gather(k)
            if k == 0:
                @pl.when(t >= 1)
                def _dr():
                    drain_scatter(3)
            else:
                drain_scatter(k - 1)

            @pl.when(ch + 3 <= NCHUNK - 1)
            def _pf():
                issue_idx((k + 3) % 4, ch + 3)

            @pl.when(ch + 2 <= NCHUNK - 1)
            def _ng():
                wait_idx((k + 2) % 4, ch + 2)
                issue_gather((k + 2) % 4)
            compute(k)
            issue_scatter(k)
        return _
    lax.fori_loop(0, NCHUNK // 4, _quad, 0)

    wait_gather(0)
    drain_scatter(3)
    compute(0)
    issue_scatter(0)
    drain_scatter(0)

    plsc.subcore_barrier()

    def _wr(k, _):
        r = row0 + k * C
        pltpu.sync_copy(agg_sh.at[pl.ds(r, C)], rows0)
        pltpu.sync_copy(rows0, agg_hbm.at[c, pl.ds(r, C)])
        return _
    lax.fori_loop(0, RPT // C, _wr, 0)
    rtail = row0 + (RPT // C) * C
    pltpu.sync_copy(agg_sh.at[pl.ds(rtail, RPT % C)],
                    rows0.at[pl.ds(0, RPT % C)])
    pltpu.sync_copy(rows0.at[pl.ds(0, RPT % C)],
                    agg_hbm.at[c, pl.ds(rtail, RPT % C)])


_SC_PARAMS = pltpu.CompilerParams(use_tc_tiling_on_sc=False,
                                  needs_layout_passes=False)


def _make_sc_a():
    mesh = plsc.VectorSubcoreMesh(core_axis_name="c", subcore_axis_name="s")
    return functools.partial(
        pl.kernel,
        out_type=[jax.ShapeDtypeStruct((NW, NCHUNK, C), jnp.float32),
                  jax.ShapeDtypeStruct((NW, N_NODES), jnp.float32)],
        mesh=mesh,
        scratch_types=[
            pltpu.VMEM((N_NODES,), jnp.float32),
            pltpu.VMEM((N_NODES,), jnp.float32),
            pltpu.VMEM((N_NODES,), jnp.float32),
            pltpu.VMEM((BPS, 2, C), jnp.int32),
            pltpu.VMEM((BPS, C), jnp.float32),
        ],
        compiler_params=_SC_PARAMS,
    )(_sc_a_body)


def _make_sc_b():
    mesh = plsc.VectorSubcoreMesh(core_axis_name="c", subcore_axis_name="s")
    return functools.partial(
        pl.kernel,
        out_type=jax.ShapeDtypeStruct((NC, N_NODES, DIM), jnp.float32),
        mesh=mesh,
        scratch_types=(
            [pltpu.VMEM((C, DIM), jnp.float32)] * 4
            + [pltpu.VMEM((2, C), jnp.int32)] * 4
            + [pltpu.VMEM((C,), jnp.float32)] * 4
            + [pltpu.VMEM_SHARED((N_NODES, DIM), jnp.float32)]
            + [pltpu.SemaphoreType.DMA] * 12
        ),
        compiler_params=_SC_PARAMS,
    )(_sc_b_body)


def kernel(node_embeddings, edge_index, W_n, a_src, a_dst):
    x = node_embeddings.astype(jnp.float32)
    eib = (edge_index.astype(jnp.int32)
           .reshape(2, NW, NCHUNK, C).transpose(1, 2, 0, 3))
    a2 = jnp.concatenate(
        [a_src.astype(jnp.float32), a_dst.astype(jnp.float32),
         jnp.zeros((DIM, 6), jnp.float32)], axis=1)

    h = pl.pallas_call(
        _tc_mm_body,
        grid=(N_NODES // MM_BLK,),
        in_specs=[
            pl.BlockSpec((MM_BLK, DIM), lambda i: (i, 0)),
            pl.BlockSpec((DIM, DIM), lambda i: (0, 0)),
        ],
        out_specs=pl.BlockSpec((MM_BLK, DIM), lambda i: (i, 0)),
        out_shape=jax.ShapeDtypeStruct((N_NODES, DIM), jnp.float32),
    )(x, W_n.astype(jnp.float32))

    s2t = pl.pallas_call(
        _tc_score_body,
        in_specs=[
            pl.BlockSpec((N_NODES, DIM), lambda: (0, 0)),
            pl.BlockSpec((DIM, 8), lambda: (0, 0)),
        ],
        out_specs=pl.BlockSpec((8, N_NODES), lambda: (0, 0)),
        out_shape=jax.ShapeDtypeStruct((8, N_NODES), jnp.float32),
    )(h, a2)

    w_e, dens = _make_sc_a()(s2t, eib)
    agg = _make_sc_b()(h, eib, w_e)

    out = pl.pallas_call(
        _tc_fin_body,
        in_specs=[
            pl.BlockSpec((NC, N_NODES, DIM), lambda: (0, 0, 0)),
            pl.BlockSpec((NW, N_NODES), lambda: (0, 0)),
        ],
        out_specs=pl.BlockSpec((N_NODES, DIM), lambda: (0, 0)),
        out_shape=jax.ShapeDtypeStruct((N_NODES, DIM), jnp.float32),
    )(agg, dens)
    return out


# speedup vs baseline: 1.1056x; 1.1056x over previous
import functools

import jax
import jax.numpy as jnp
from jax import lax
from jax.experimental import pallas as pl
from jax.experimental.pallas import tpu as pltpu
from jax.experimental.pallas import tpu_sc as plsc

N_NODES = 10000
N_EDGES = 320000
DIM = 128

NC = 2
NS = 16
NW = NC * NS
EPT = N_EDGES // NW
C = 80
NCHUNK = EPT // C
RPT = N_NODES // NS
ZB = 125

MM_BLK = 1000


def _tc_mm_body(x_ref, w_ref, h_ref):
    h_ref[...] = jnp.dot(x_ref[...], w_ref[...],
                         preferred_element_type=jnp.float32)


def _tc_score_body(h_ref, a2_ref, s_ref):
    s_ref[...] = lax.dot_general(a2_ref[...], h_ref[...],
                                 (((0,), (1,)), ((), ())),
                                 preferred_element_type=jnp.float32)


def _tc_fin_body(agg_ref, den_ref, o_ref):
    a = agg_ref[0] + agg_ref[1]
    dsum = jnp.sum(den_ref[...], axis=0)
    o_ref[...] = jnp.maximum(a / (dsum[:, None] + 1e-16), 0.0)


ASC = 5
BPS = NCHUNK // ASC


def _sc_a_body(s2t_hbm, eia_hbm, w_hbm, den_hbm,
               ssrc_v, sdst_v, den_v, eidx_v, wv_v):
    c = lax.axis_index("c")
    s = lax.axis_index("s")
    wid = s * NC + c

    pltpu.sync_copy(s2t_hbm.at[0], ssrc_v)
    pltpu.sync_copy(s2t_hbm.at[1], sdst_v)

    def _zden(i, _):
        den_v[pl.ds(i * 16, 16)] = jnp.zeros((16,), jnp.float32)
        return _
    lax.fori_loop(0, N_NODES // 16, _zden, 0)

    def _super(t, _):
        pltpu.sync_copy(eia_hbm.at[wid, pl.ds(t * BPS, BPS)], eidx_v)

        def _cc(cc, _c):
            def _grp(g, _g):
                base = g * 16
                s16 = eidx_v[cc, 0, pl.ds(base, 16)]
                d16 = eidx_v[cc, 1, pl.ds(base, 16)]
                x = (plsc.load_gather(ssrc_v, [s16])
                     + plsc.load_gather(sdst_v, [d16]))
                e = jnp.where(x >= 0, x, 0.2 * x)
                w16 = jnp.exp(e)
                wv_v[cc, pl.ds(base, 16)] = w16
                plsc.addupdate_scatter(den_v, [d16], w16)
                return _g
            lax.fori_loop(0, C // 16, _grp, 0)
            return _c
        lax.fori_loop(0, BPS, _cc, 0)
        pltpu.sync_copy(wv_v, w_hbm.at[wid, pl.ds(t * BPS, BPS)])
        return _
    lax.fori_loop(0, ASC, _super, 0)
    pltpu.sync_copy(den_v, den_hbm.at[wid])


def _sc_b_body(h_hbm, eib_hbm, wb_hbm, agg_hbm,
               rows0, rows1, rows2, rows3, idx0, idx1, idx2, idx3,
               wv0, wv1, wv2, wv3, agg_sh,
               g0, g1, g2, g3, i0, i1, i2, i3, s0, s1, s2, s3):
    c = lax.axis_index("c")
    s = lax.axis_index("s")
    wid = s * NC + c
    row0 = s * RPT

    rows = (rows0, rows1, rows2, rows3)
    idx = (idx0, idx1, idx2, idx3)
    wv = (wv0, wv1, wv2, wv3)
    gsem = (g0, g1, g2, g3)
    isem = (i0, i1, i2, i3)
    ssem = (s0, s1, s2, s3)

    def issue_idx(b, ch):
        pltpu.async_copy(eib_hbm.at[wid, ch], idx[b], isem[b])
        pltpu.async_copy(wb_hbm.at[wid, ch], wv[b], isem[b])

    def wait_idx(b, ch):
        pltpu.make_async_copy(eib_hbm.at[wid, ch], idx[b], isem[b]).wait()
        pltpu.make_async_copy(wb_hbm.at[wid, ch], wv[b], isem[b]).wait()

    def issue_gather(b):
        pltpu.async_copy(h_hbm.at[idx[b].at[0]], rows[b], gsem[b])

    def wait_gather(b):
        pltpu.make_async_copy(h_hbm.at[idx[b].at[0]], rows[b],
                              gsem[b]).wait()

    def issue_scatter(b):
        pltpu.async_copy(rows[b], agg_sh.at[idx[b].at[1]], ssem[b],
                         add=True)

    def drain_scatter(b):
        pltpu.make_async_copy(rows[b], agg_sh.at[idx[b].at[1]],
                              ssem[b]).wait()

    def compute(b):
        def _grp(g, _g):
            base = g * 16
            w16 = wv[b][pl.ds(base, 16)]
            for l in range(16):
                wl = w16[l]
                for j in range(DIM // 16):
                    sl = pl.ds(j * 16, 16)
                    rows[b][base + l, sl] = rows[b][base + l, sl] * wl
            return _g
        lax.fori_loop(0, C // 16, _grp, 0)

    def _zrows(i, _):
        for j in range(DIM // 16):
            rows0[i, pl.ds(j * 16, 16)] = jnp.zeros((16,), jnp.float32)
        return _
    lax.fori_loop(0, C, _zrows, 0)

    def _zagg(k, _):
        pltpu.sync_copy(rows0, agg_sh.at[pl.ds(row0 + k * C, C)])
        return _
    lax.fori_loop(0, RPT // C, _zagg, 0)
    pltpu.sync_copy(rows0.at[pl.ds(0, RPT % C)],
                    agg_sh.at[pl.ds(row0 + (RPT // C) * C, RPT % C)])
    plsc.subcore_barrier()

    pltpu.sync_copy(eib_hbm.at[wid, 0], idx0)
    pltpu.sync_copy(wb_hbm.at[wid, 0], wv0)
    issue_gather(0)
    issue_idx(1, 1)
    issue_idx(2, 2)
    wait_idx(1, 1)
    issue_gather(1)

    def _quad(t, _):
        for k in range(4):
            ch = t * 4 + k
            wait_gather(k)
            if k == 0:
                @pl.when(t >= 1)
                def _dr():
                    drain_scatter(3)
            else:
                drain_scatter(k - 1)

            @pl.when(ch + 3 <= NCHUNK - 1)
            def _pf():
                issue_idx((k + 3) % 4, ch + 3)

            @pl.when(ch + 2 <= NCHUNK - 1)
            def _ng():
                wait_idx((k + 2) % 4, ch + 2)
                issue_gather((k + 2) % 4)
            compute(k)
            issue_scatter(k)
        return _
    lax.fori_loop(0, NCHUNK // 4, _quad, 0)

    wait_gather(0)
    drain_scatter(3)
    compute(0)
    issue_scatter(0)
    drain_scatter(0)

    plsc.subcore_barrier()

    def _wr(k, _):
        r = row0 + k * C
        pltpu.sync_copy(agg_sh.at[pl.ds(r, C)], rows0)
        pltpu.sync_copy(rows0, agg_hbm.at[c, pl.ds(r, C)])
        return _
    lax.fori_loop(0, RPT // C, _wr, 0)
    rtail = row0 + (RPT // C) * C
    pltpu.sync_copy(agg_sh.at[pl.ds(rtail, RPT % C)],
                    rows0.at[pl.ds(0, RPT % C)])
    pltpu.sync_copy(rows0.at[pl.ds(0, RPT % C)],
                    agg_hbm.at[c, pl.ds(rtail, RPT % C)])


_SC_PARAMS = pltpu.CompilerParams(use_tc_tiling_on_sc=False,
                                  needs_layout_passes=False)


def _make_sc_a():
    mesh = plsc.VectorSubcoreMesh(core_axis_name="c", subcore_axis_name="s")
    return functools.partial(
        pl.kernel,
        out_type=[jax.ShapeDtypeStruct((NW, NCHUNK, C), jnp.float32),
                  jax.ShapeDtypeStruct((NW, N_NODES), jnp.float32)],
        mesh=mesh,
        scratch_types=[
            pltpu.VMEM((N_NODES,), jnp.float32),
            pltpu.VMEM((N_NODES,), jnp.float32),
            pltpu.VMEM((N_NODES,), jnp.float32),
            pltpu.VMEM((BPS, 2, C), jnp.int32),
            pltpu.VMEM((BPS, C), jnp.float32),
        ],
        compiler_params=_SC_PARAMS,
    )(_sc_a_body)


def _make_sc_b():
    mesh = plsc.VectorSubcoreMesh(core_axis_name="c", subcore_axis_name="s")
    return functools.partial(
        pl.kernel,
        out_type=jax.ShapeDtypeStruct((NC, N_NODES, DIM), jnp.float32),
        mesh=mesh,
        scratch_types=(
            [pltpu.VMEM((C, DIM), jnp.float32)] * 4
            + [pltpu.VMEM((2, C), jnp.int32)] * 4
            + [pltpu.VMEM((C,), jnp.float32)] * 4
            + [pltpu.VMEM_SHARED((N_NODES, DIM), jnp.float32)]
            + [pltpu.SemaphoreType.DMA] * 12
        ),
        compiler_params=_SC_PARAMS,
    )(_sc_b_body)


def kernel(node_embeddings, edge_index, W_n, a_src, a_dst):
    x = node_embeddings.astype(jnp.float32)
    eib = (edge_index.astype(jnp.int32)
           .reshape(2, NW, NCHUNK, C).transpose(1, 2, 0, 3))
    a2 = jnp.concatenate(
        [a_src.astype(jnp.float32), a_dst.astype(jnp.float32),
         jnp.zeros((DIM, 6), jnp.float32)], axis=1)

    h = pl.pallas_call(
        _tc_mm_body,
        grid=(N_NODES // MM_BLK,),
        in_specs=[
            pl.BlockSpec((MM_BLK, DIM), lambda i: (i, 0)),
            pl.BlockSpec((DIM, DIM), lambda i: (0, 0)),
        ],
        out_specs=pl.BlockSpec((MM_BLK, DIM), lambda i: (i, 0)),
        out_shape=jax.ShapeDtypeStruct((N_NODES, DIM), jnp.float32),
    )(x, W_n.astype(jnp.float32))

    s2t = pl.pallas_call(
        _tc_score_body,
        in_specs=[
            pl.BlockSpec((N_NODES, DIM), lambda: (0, 0)),
            pl.BlockSpec((DIM, 8), lambda: (0, 0)),
        ],
        out_specs=pl.BlockSpec((8, N_NODES), lambda: (0, 0)),
        out_shape=jax.ShapeDtypeStruct((8, N_NODES), jnp.float32),
    )(h, a2)

    w_e, dens = _make_sc_a()(s2t, eib)
    agg = _make_sc_b()(h, eib, w_e)

    out = pl.pallas_call(
        _tc_fin_body,
        in_specs=[
            pl.BlockSpec((NC, N_NODES, DIM), lambda: (0, 0, 0)),
            pl.BlockSpec((NW, N_NODES), lambda: (0, 0)),
        ],
        out_specs=pl.BlockSpec((N_NODES, DIM), lambda: (0, 0)),
        out_shape=jax.ShapeDtypeStruct((N_NODES, DIM), jnp.float32),
    )(agg, dens)
    return out


# speedup vs baseline: 1.1481x; 1.0385x over previous
import functools

import jax
import jax.numpy as jnp
from jax import lax
from jax.experimental import pallas as pl
from jax.experimental.pallas import tpu as pltpu
from jax.experimental.pallas import tpu_sc as plsc

N_NODES = 10000
N_EDGES = 320000
DIM = 128

NC = 2
NS = 16
NW = NC * NS
EPT = N_EDGES // NW
C = 80
NCHUNK = EPT // C
RPT = N_NODES // NS
ZB = 125

MM_BLK = 1000


def _tc_mm_body(x_ref, w_ref, h_ref):
    h_ref[...] = jnp.dot(x_ref[...], w_ref[...],
                         preferred_element_type=jnp.float32)


def _tc_score_body(h_ref, a2_ref, s_ref):
    s_ref[...] = lax.dot_general(a2_ref[...], h_ref[...],
                                 (((0,), (1,)), ((), ())),
                                 preferred_element_type=jnp.float32)


def _tc_fin_body(agg_ref, den_ref, o_ref):
    a = agg_ref[0] + agg_ref[1]
    dsum = jnp.sum(den_ref[...], axis=0)
    o_ref[...] = jnp.maximum(a / (dsum[:, None] + 1e-16), 0.0)


ASC = 5
AE = EPT // ASC


def _sc_a_body(s2t_hbm, eia_hbm, w_hbm, den_hbm,
               ssrc_v, sdst_v, den_v, eidx_v, wv_v):
    c = lax.axis_index("c")
    s = lax.axis_index("s")
    wid = s * NC + c

    pltpu.sync_copy(s2t_hbm.at[0], ssrc_v)
    pltpu.sync_copy(s2t_hbm.at[1], sdst_v)

    def _zden(i, _):
        den_v[pl.ds(i * 16, 16)] = jnp.zeros((16,), jnp.float32)
        return _
    lax.fori_loop(0, N_NODES // 16, _zden, 0)

    def _super(t, _):
        pltpu.sync_copy(eia_hbm.at[wid, t], eidx_v)

        def _grp(g, _g):
            base = g * 16
            s16 = eidx_v[0, pl.ds(base, 16)]
            d16 = eidx_v[1, pl.ds(base, 16)]
            x = (plsc.load_gather(ssrc_v, [s16])
                 + plsc.load_gather(sdst_v, [d16]))
            e = jnp.where(x >= 0, x, 0.2 * x)
            w16 = jnp.exp(e)
            wv_v[pl.ds(base, 16)] = w16
            plsc.addupdate_scatter(den_v, [d16], w16)
            return _g
        lax.fori_loop(0, AE // 16, _grp, 0)
        pltpu.sync_copy(wv_v, w_hbm.at[wid, t])
        return _
    lax.fori_loop(0, ASC, _super, 0)
    pltpu.sync_copy(den_v, den_hbm.at[wid])


def _sc_b_body(h_hbm, eib_hbm, wb_hbm, agg_hbm,
               rows0, rows1, rows2, rows3, idx0, idx1, idx2, idx3,
               wv0, wv1, wv2, wv3, agg_sh,
               g0, g1, g2, g3, i0, i1, i2, i3, s0, s1, s2, s3):
    c = lax.axis_index("c")
    s = lax.axis_index("s")
    wid = s * NC + c
    row0 = s * RPT

    rows = (rows0, rows1, rows2, rows3)
    idx = (idx0, idx1, idx2, idx3)
    wv = (wv0, wv1, wv2, wv3)
    gsem = (g0, g1, g2, g3)
    isem = (i0, i1, i2, i3)
    ssem = (s0, s1, s2, s3)

    def issue_idx(b, ch):
        pltpu.async_copy(eib_hbm.at[wid, ch], idx[b], isem[b])
        pltpu.async_copy(wb_hbm.at[wid, ch], wv[b], isem[b])

    def wait_idx(b, ch):
        pltpu.make_async_copy(eib_hbm.at[wid, ch], idx[b], isem[b]).wait()
        pltpu.make_async_copy(wb_hbm.at[wid, ch], wv[b], isem[b]).wait()

    def issue_gather(b):
        pltpu.async_copy(h_hbm.at[idx[b].at[0]], rows[b], gsem[b])

    def wait_gather(b):
        pltpu.make_async_copy(h_hbm.at[idx[b].at[0]], rows[b],
                              gsem[b]).wait()

    def issue_scatter(b):
        pltpu.async_copy(rows[b], agg_sh.at[idx[b].at[1]], ssem[b],
                         add=True)

    def drain_scatter(b):
        pltpu.make_async_copy(rows[b], agg_sh.at[idx[b].at[1]],
                              ssem[b]).wait()

    def compute(b):
        def _grp(g, _g):
            base = g * 16
            w16 = wv[b][pl.ds(base, 16)]
            for l in range(16):
                wl = w16[l]
                for j in range(DIM // 16):
                    sl = pl.ds(j * 16, 16)
                    rows[b][base + l, sl] = rows[b][base + l, sl] * wl
            return _g
        lax.fori_loop(0, C // 16, _grp, 0)

    def _zrows(i, _):
        for j in range(DIM // 16):
            rows0[i, pl.ds(j * 16, 16)] = jnp.zeros((16,), jnp.float32)
        return _
    lax.fori_loop(0, C, _zrows, 0)

    def _zagg(k, _):
        pltpu.sync_copy(rows0, agg_sh.at[pl.ds(row0 + k * C, C)])
        return _
    lax.fori_loop(0, RPT // C, _zagg, 0)
    pltpu.sync_copy(rows0.at[pl.ds(0, RPT % C)],
                    agg_sh.at[pl.ds(row0 + (RPT // C) * C, RPT % C)])
    plsc.subcore_barrier()

    pltpu.sync_copy(eib_hbm.at[wid, 0], idx0)
    pltpu.sync_copy(wb_hbm.at[wid, 0], wv0)
    issue_gather(0)
    issue_idx(1, 1)
    issue_idx(2, 2)
    wait_idx(1, 1)
    issue_gather(1)

    def _quad(t, _):
        for k in range(4):
            ch = t * 4 + k
            wait_gather(k)
            if k == 0:
                @pl.when(t >= 1)
                def _dr():
                    drain_scatter(3)
            else:
                drain_scatter(k - 1)

            @pl.when(ch + 3 <= NCHUNK - 1)
            def _pf():
                issue_idx((k + 3) % 4, ch + 3)

            @pl.when(ch + 2 <= NCHUNK - 1)
            def _ng():
                wait_idx((k + 2) % 4, ch + 2)
                issue_gather((k + 2) % 4)
            compute(k)
            issue_scatter(k)
        return _
    lax.fori_loop(0, NCHUNK // 4, _quad, 0)

    wait_gather(0)
    drain_scatter(3)
    compute(0)
    issue_scatter(0)
    drain_scatter(0)

    plsc.subcore_barrier()

    def _wr(k, _):
        r = row0 + k * C
        pltpu.sync_copy(agg_sh.at[pl.ds(r, C)], rows0)
        pltpu.sync_copy(rows0, agg_hbm.at[c, pl.ds(r, C)])
        return _
    lax.fori_loop(0, RPT // C, _wr, 0)
    rtail = row0 + (RPT // C) * C
    pltpu.sync_copy(agg_sh.at[pl.ds(rtail, RPT % C)],
                    rows0.at[pl.ds(0, RPT % C)])
    pltpu.sync_copy(rows0.at[pl.ds(0, RPT % C)],
                    agg_hbm.at[c, pl.ds(rtail, RPT % C)])


_SC_PARAMS = pltpu.CompilerParams(use_tc_tiling_on_sc=False,
                                  needs_layout_passes=False)


def _make_sc_a():
    mesh = plsc.VectorSubcoreMesh(core_axis_name="c", subcore_axis_name="s")
    return functools.partial(
        pl.kernel,
        out_type=[jax.ShapeDtypeStruct((NW, ASC, AE), jnp.float32),
                  jax.ShapeDtypeStruct((NW, N_NODES), jnp.float32)],
        mesh=mesh,
        scratch_types=[
            pltpu.VMEM((N_NODES,), jnp.float32),
            pltpu.VMEM((N_NODES,), jnp.float32),
            pltpu.VMEM((N_NODES,), jnp.float32),
            pltpu.VMEM((2, AE), jnp.int32),
            pltpu.VMEM((AE,), jnp.float32),
        ],
        compiler_params=_SC_PARAMS,
    )(_sc_a_body)


def _make_sc_b():
    mesh = plsc.VectorSubcoreMesh(core_axis_name="c", subcore_axis_name="s")
    return functools.partial(
        pl.kernel,
        out_type=jax.ShapeDtypeStruct((NC, N_NODES, DIM), jnp.float32),
        mesh=mesh,
        scratch_types=(
            [pltpu.VMEM((C, DIM), jnp.float32)] * 4
            + [pltpu.VMEM((2, C), jnp.int32)] * 4
            + [pltpu.VMEM((C,), jnp.float32)] * 4
            + [pltpu.VMEM_SHARED((N_NODES, DIM), jnp.float32)]
            + [pltpu.SemaphoreType.DMA] * 12
        ),
        compiler_params=_SC_PARAMS,
    )(_sc_b_body)


def kernel(node_embeddings, edge_index, W_n, a_src, a_dst):
    x = node_embeddings.astype(jnp.float32)
    ei32 = edge_index.astype(jnp.int32)
    eia = ei32.reshape(2, NW, ASC, AE).transpose(1, 2, 0, 3)
    eib = ei32.reshape(2, NW, NCHUNK, C).transpose(1, 2, 0, 3)
    a2 = jnp.concatenate(
        [a_src.astype(jnp.float32), a_dst.astype(jnp.float32),
         jnp.zeros((DIM, 6), jnp.float32)], axis=1)

    h = pl.pallas_call(
        _tc_mm_body,
        grid=(N_NODES // MM_BLK,),
        in_specs=[
            pl.BlockSpec((MM_BLK, DIM), lambda i: (i, 0)),
            pl.BlockSpec((DIM, DIM), lambda i: (0, 0)),
        ],
        out_specs=pl.BlockSpec((MM_BLK, DIM), lambda i: (i, 0)),
        out_shape=jax.ShapeDtypeStruct((N_NODES, DIM), jnp.float32),
    )(x, W_n.astype(jnp.float32))

    s2t = pl.pallas_call(
        _tc_score_body,
        in_specs=[
            pl.BlockSpec((N_NODES, DIM), lambda: (0, 0)),
            pl.BlockSpec((DIM, 8), lambda: (0, 0)),
        ],
        out_specs=pl.BlockSpec((8, N_NODES), lambda: (0, 0)),
        out_shape=jax.ShapeDtypeStruct((8, N_NODES), jnp.float32),
    )(h, a2)

    w_e, dens = _make_sc_a()(s2t, eia)
    agg = _make_sc_b()(h, eib, w_e.reshape(NW, NCHUNK, C))

    out = pl.pallas_call(
        _tc_fin_body,
        in_specs=[
            pl.BlockSpec((NC, N_NODES, DIM), lambda: (0, 0, 0)),
            pl.BlockSpec((NW, N_NODES), lambda: (0, 0)),
        ],
        out_specs=pl.BlockSpec((N_NODES, DIM), lambda: (0, 0)),
        out_shape=jax.ShapeDtypeStruct((N_NODES, DIM), jnp.float32),
    )(agg, dens)
    return out
